# speedup vs baseline: 33.2780x; 33.2780x over previous
import numpy as np

import jax
import jax.numpy as jnp
from jax.experimental import pallas as pl
from jax.experimental.pallas import tpu as pltpu

_CAP = 8
_LAT = 16
_H_IN = 28
_KS, _STRIDE, _PAD = 4, 2, 1
_H1 = (_H_IN + 2 * _PAD - _KS) // _STRIDE + 1
_H2 = (_H1 + 2 * _PAD - _KS) // _STRIDE + 1
_KK = _KS * _KS
_P1 = _H1 * _H1
_P2 = _H2 * _H2
_C1 = _CAP
_C2 = 2 * _CAP
_D_IN = _H_IN * _H_IN
_F1 = _C1 * _P1
_F2 = _C2 * _P2
_NOUT = 2 * _LAT


def _build_sel1():
    sel = np.zeros((_KK, _D_IN, _P1), np.float32)
    for kh in range(_KS):
        for kw in range(_KS):
            t = kh * _KS + kw
            for oh in range(_H1):
                for ow in range(_H1):
                    ih = oh * _STRIDE + kh - _PAD
                    iw = ow * _STRIDE + kw - _PAD
                    if 0 <= ih < _H_IN and 0 <= iw < _H_IN:
                        sel[t, ih * _H_IN + iw, oh * _H1 + ow] = 1.0
    return sel


_SEL1 = _build_sel1()


def _enc_kernel(x_ref, a1_ref, b1_ref, a2_ref, b2_ref, wfc_ref, bfc_ref,
                o_ref):
    xb = x_ref[...].astype(jnp.bfloat16)
    y1 = jnp.dot(xb, a1_ref[...], preferred_element_type=jnp.float32)
    y1 = jnp.maximum(y1 + b1_ref[...], 0.0).astype(jnp.bfloat16)
    y2 = jnp.dot(y1, a2_ref[...], preferred_element_type=jnp.float32)
    y2 = jnp.maximum(y2 + b2_ref[...], 0.0).astype(jnp.bfloat16)
    o_ref[...] = (jnp.dot(y2, wfc_ref[...],
                          preferred_element_type=jnp.float32) + bfc_ref[...])


def kernel(x, w1t, b1, w2t, b2, wfc3, bfc, sel):
    N = x.shape[0]
    xf = x.reshape(N, _D_IN)

    sel1 = jnp.asarray(_SEL1)
    a1 = jnp.einsum('ct,tdp->dcp', w1t, sel1).reshape(_D_IN, _F1)
    w2r = w2t.reshape(_C2, _KK, _C1)
    a2 = jnp.einsum('ktc,tpq->cpkq', w2r, sel).reshape(_F1, _F2)
    wfc = wfc3.reshape(_F2, _NOUT)
    b1r = jnp.repeat(b1.reshape(_C1), _P1).reshape(1, _F1)
    b2r = jnp.repeat(b2.reshape(_C2), _P2).reshape(1, _F2)

    a1 = a1.astype(jnp.bfloat16)
    a2 = a2.astype(jnp.bfloat16)
    wfc = wfc.astype(jnp.bfloat16)

    B = 512
    out = pl.pallas_call(
        _enc_kernel,
        out_shape=jax.ShapeDtypeStruct((N, _NOUT), jnp.float32),
        grid=(N // B,),
        in_specs=[
            pl.BlockSpec((B, _D_IN), lambda i: (i, 0)),
            pl.BlockSpec((_D_IN, _F1), lambda i: (0, 0)),
            pl.BlockSpec((1, _F1), lambda i: (0, 0)),
            pl.BlockSpec((_F1, _F2), lambda i: (0, 0)),
            pl.BlockSpec((1, _F2), lambda i: (0, 0)),
            pl.BlockSpec((_F2, _NOUT), lambda i: (0, 0)),
            pl.BlockSpec((1, _NOUT), lambda i: (0, 0)),
        ],
        out_specs=pl.BlockSpec((B, _NOUT), lambda i: (i, 0)),
        compiler_params=pltpu.CompilerParams(
            dimension_semantics=("parallel",)),
    )(xf, a1, b1r, a2, b2r, wfc, bfc)

    return out[:, :_LAT], out[:, _LAT:]
